# baseline (speedup 1.0000x reference)
import functools

import jax
import jax.numpy as jnp
from jax import lax
from jax.experimental import pallas as pl
from jax.experimental.pallas import tpu as pltpu
from jax.experimental.pallas import tpu_sc as plsc

N_ROWS = 320000
D = 128
DC = D // 16
S = 10000
NC, NS = 2, 16
NWORK = NC * NS
SEG_PER_W = 312
SEG_LAST = 320
ACC_R = 5024
W = 64
NB = 8
LA = 4
SW = 8
SCALE = 1.0 / (32.0 ** 0.5)


def _segsum(x, idx2d, idx1d, bounds):
    mesh = plsc.VectorSubcoreMesh(core_axis_name="c", subcore_axis_name="s")

    @functools.partial(
        pl.kernel,
        out_type=jax.ShapeDtypeStruct((S, D), jnp.float32),
        mesh=mesh,
        scratch_types=[
            pltpu.VMEM((48,), jnp.int32),
            pltpu.VMEM((2, SW, W), jnp.int32),
            pltpu.VMEM((W,), jnp.int32),
            pltpu.VMEM((NB, W, D), jnp.float32),
            pltpu.VMEM((64, D), jnp.float32),
            pltpu.VMEM_SHARED((ACC_R, D), jnp.float32),
            pltpu.SemaphoreType.DMA((2,)),
            pltpu.SemaphoreType.DMA((NB,)),
            pltpu.SemaphoreType.DMA((NB,)),
        ],
    )
    def k(x_hbm, idx2d_hbm, idx1d_hbm, bounds_hbm, out_hbm, bounds_v, idx_sv,
          idx_r, rows_v, zbuf, acc, idx_sem, load_sem, scat_sem):
        c = lax.axis_index("c")
        s = lax.axis_index("s")
        wid = c * NS + s
        seg0 = c * 5000 + s * SEG_PER_W

        pltpu.sync_copy(bounds_hbm, bounds_v)
        bv = bounds_v[pl.ds(wid, 16)]
        start = bv[0]
        end = bv[1]
        a64 = pl.multiple_of((start // (SW * W)) * SW, 8)
        astart = pl.multiple_of(a64 * W, SW * W)
        nw = (end - astart) // W
        nsup = (nw + (SW - 1)) // SW

        zero = jnp.zeros((16,), jnp.float32)

        def zero_body(i, carry):
            zbuf[i // DC, pl.ds((i % DC) * 16, 16)] = zero
            return carry

        lax.fori_loop(0, 64 * DC, zero_body, 0)
        arow = s * SEG_PER_W
        for off, size in ((0, 64), (64, 64), (128, 64), (192, 64), (256, 56)):
            pltpu.sync_copy(zbuf.at[pl.ds(0, size), :],
                            acc.at[pl.ds(arow + off, size), :])

        @pl.when(s == NS - 1)
        def _zero_last():
            pltpu.sync_copy(zbuf.at[pl.ds(0, SEG_LAST - SEG_PER_W), :],
                            acc.at[pl.ds(arow + SEG_PER_W,
                                         SEG_LAST - SEG_PER_W), :])

        def _issue_sup(h, j):
            pltpu.async_copy(idx2d_hbm.at[pl.ds(a64 + j * SW, SW), :],
                             idx_sv.at[h], idx_sem.at[h])

        def _wait_sup(h, j):
            pltpu.make_async_copy(idx2d_hbm.at[pl.ds(a64 + j * SW, SW), :],
                                  idx_sv.at[h], idx_sem.at[h]).wait()

        def _issue_load(b, k_win):
            r0 = astart + k_win * W
            pltpu.async_copy(x_hbm.at[pl.ds(r0, W), :], rows_v.at[b],
                             load_sem.at[b])

        def _wait_load(b, k_win):
            r0 = astart + k_win * W
            pltpu.make_async_copy(x_hbm.at[pl.ds(r0, W), :], rows_v.at[b],
                                  load_sem.at[b]).wait()

        def _wait_scat(b):
            pltpu.make_async_copy(rows_v.at[b], acc.at[idx_r],
                                  scat_sem.at[b]).wait()

        def _localize(iref, w0, lo, hi):
            def group(g, carry):
                r16 = g * 16
                rgv = w0 + r16 + lax.iota(jnp.int32, 16)
                inb = jnp.logical_and(rgv >= lo, rgv < hi)
                tgtv = jnp.where(inb, iref[pl.ds(r16, 16)] - c * 5000,
                                 5000 + s)
                iref[pl.ds(r16, 16)] = tgtv
                return carry

            lax.fori_loop(0, W // 16, group, 0)

        for h in range(2):
            @pl.when(h < nsup)
            def _():
                _issue_sup(h, h)

        for b in range(LA):
            @pl.when(b < nw)
            def _():
                _issue_load(b, b)

        def outer(t2, carry):
            for half in range(2):
                t = t2 * 2 + half

                @pl.when(t < nsup)
                def _():
                    _wait_sup(half, t)
                    for b in range(NB):
                        k_win = t * SW + b

                        @pl.when(k_win < nw)
                        def _():
                            _wait_load(b, k_win)
                            _localize(idx_sv.at[half, b], astart + k_win * W,
                                      start, end)
                            pltpu.async_copy(rows_v.at[b],
                                             acc.at[idx_sv.at[half, b]],
                                             scat_sem.at[b], add=True)
                            pb = (b + LA) % NB

                            @pl.when(k_win >= LA)
                            def _():
                                _wait_scat(pb)

                            @pl.when(k_win + LA < nw)
                            def _():
                                _issue_load(pb, k_win + LA)

                    @pl.when(t + 2 < nsup)
                    def _():
                        _issue_sup(half, t + 2)
            return carry

        lax.fori_loop(0, (nsup + 1) // 2, outer, 0)

        for b in range(NB):
            cond = jnp.zeros((), jnp.bool_)
            for dt in range(1, LA + 1):
                jt = nw - dt
                cond = jnp.logical_or(
                    cond, jnp.logical_and(jt >= 0, jt % NB == b))

            @pl.when(cond)
            def _():
                _wait_scat(b)

        rem = end - (astart + nw * W)

        @pl.when(rem > 0)
        def _remainder():
            r1 = jnp.maximum(((end + 7) // 8) * 8 - W, 0)
            r1 = pl.multiple_of(r1, 8)
            pltpu.sync_copy(idx1d_hbm.at[pl.ds(r1, W)], idx_r)
            pltpu.sync_copy(x_hbm.at[pl.ds(r1, W), :], rows_v.at[0])
            _localize(idx_r, r1, jnp.maximum(start, astart + nw * W), end)
            pltpu.sync_copy(rows_v.at[0], acc.at[idx_r], add=True)

        def _scale_out(aoff, size):
            pltpu.sync_copy(acc.at[pl.ds(arow + aoff, size), :],
                            zbuf.at[pl.ds(0, size), :])

            def scale_body(i, carry):
                zbuf[i // DC, pl.ds((i % DC) * 16, 16)] = (
                    zbuf[i // DC, pl.ds((i % DC) * 16, 16)] * SCALE)
                return carry

            lax.fori_loop(0, size * DC, scale_body, 0)
            pltpu.sync_copy(zbuf.at[pl.ds(0, size), :],
                            out_hbm.at[pl.ds(seg0 + aoff, size), :])

        for off, size in ((0, 64), (64, 64), (128, 64), (192, 64), (256, 56)):
            _scale_out(off, size)

        @pl.when(s == NS - 1)
        def _last():
            _scale_out(SEG_PER_W, SEG_LAST - SEG_PER_W)

    return k(x, idx2d, idx1d, bounds)


def kernel(x, index, dim, dim_size):
    del dim, dim_size
    idx = index.astype(jnp.int32)
    wids = jnp.arange(NWORK, dtype=jnp.int32)
    targets = (wids // NS) * 5000 + (wids % NS) * SEG_PER_W
    bounds = jnp.searchsorted(idx, targets).astype(jnp.int32)
    bounds = jnp.concatenate(
        [bounds, jnp.full((48 - NWORK,), N_ROWS, dtype=jnp.int32)])
    idx2d = jnp.concatenate(
        [idx, jnp.zeros((SW * W,), jnp.int32)]).reshape(N_ROWS // W + SW, W)
    return _segsum(x, idx2d, idx, bounds)


# speedup vs baseline: 1.0402x; 1.0402x over previous
import functools

import jax
import jax.numpy as jnp
from jax import lax
from jax.experimental import pallas as pl
from jax.experimental.pallas import tpu as pltpu
from jax.experimental.pallas import tpu_sc as plsc

N_ROWS = 320000
D = 128
DC = D // 16
S = 10000
NC, NS = 2, 16
NWORK = NC * NS
SEG_PER_W = 312
SEG_LAST = 320
ACC_R = 5024
W = 64
NB = 8
LA = 4
SCALE = 1.0 / (32.0 ** 0.5)


def _segsum(x, idx1d, bounds):
    mesh = plsc.VectorSubcoreMesh(core_axis_name="c", subcore_axis_name="s")

    @functools.partial(
        pl.kernel,
        out_type=jax.ShapeDtypeStruct((S, D), jnp.float32),
        mesh=mesh,
        scratch_types=[
            pltpu.VMEM((48,), jnp.int32),
            pltpu.VMEM((NB, W), jnp.int32),
            pltpu.VMEM((W,), jnp.int32),
            pltpu.VMEM((NB, W, D), jnp.float32),
            pltpu.VMEM((64, D), jnp.float32),
            pltpu.VMEM_SHARED((ACC_R, D), jnp.float32),
            pltpu.SemaphoreType.DMA((NB,)),
            pltpu.SemaphoreType.DMA((NB,)),
        ],
    )
    def k(x_hbm, idx1d_hbm, bounds_hbm, out_hbm, bounds_v, idx_v,
          idx_r, rows_v, zbuf, acc, load_sem, scat_sem):
        c = lax.axis_index("c")
        s = lax.axis_index("s")
        wid = c * NS + s
        seg0 = c * 5000 + s * SEG_PER_W

        pltpu.sync_copy(bounds_hbm, bounds_v)
        bv = bounds_v[pl.ds(wid, 16)]
        start = bv[0]
        end = bv[1]
        astart = pl.multiple_of((start // W) * W, W)
        nw = (end - astart) // W

        zero = jnp.zeros((16,), jnp.float32)

        def zero_body(i, carry):
            zbuf[i // DC, pl.ds((i % DC) * 16, 16)] = zero
            return carry

        lax.fori_loop(0, 64 * DC, zero_body, 0)
        arow = s * SEG_PER_W
        for off, size in ((0, 64), (64, 64), (128, 64), (192, 64), (256, 56)):
            pltpu.sync_copy(zbuf.at[pl.ds(0, size), :],
                            acc.at[pl.ds(arow + off, size), :])

        @pl.when(s == NS - 1)
        def _zero_last():
            pltpu.sync_copy(zbuf.at[pl.ds(0, SEG_LAST - SEG_PER_W), :],
                            acc.at[pl.ds(arow + SEG_PER_W,
                                         SEG_LAST - SEG_PER_W), :])

        def _issue_load(b, k_win):
            r0 = astart + k_win * W
            pltpu.async_copy(idx1d_hbm.at[pl.ds(r0, W)], idx_v.at[b],
                             load_sem.at[b])
            pltpu.async_copy(x_hbm.at[pl.ds(r0, W), :], rows_v.at[b],
                             load_sem.at[b])

        def _wait_load(b, k_win):
            r0 = astart + k_win * W
            pltpu.make_async_copy(idx1d_hbm.at[pl.ds(r0, W)], idx_v.at[b],
                                  load_sem.at[b]).wait()
            pltpu.make_async_copy(x_hbm.at[pl.ds(r0, W), :], rows_v.at[b],
                                  load_sem.at[b]).wait()

        def _wait_scat(b):
            pltpu.make_async_copy(rows_v.at[b], acc.at[idx_r],
                                  scat_sem.at[b]).wait()

        def _localize(iref, w0, lo, hi):
            def group(g, carry):
                r16 = g * 16
                rgv = w0 + r16 + lax.iota(jnp.int32, 16)
                inb = jnp.logical_and(rgv >= lo, rgv < hi)
                tgtv = jnp.where(inb, iref[pl.ds(r16, 16)] - c * 5000,
                                 5000 + s)
                iref[pl.ds(r16, 16)] = tgtv
                return carry

            lax.fori_loop(0, W // 16, group, 0)

        for b in range(LA):
            @pl.when(b < nw)
            def _():
                _issue_load(b, b)

        def outer(t, carry):
            for b in range(NB):
                k_win = t * NB + b

                @pl.when(k_win < nw)
                def _():
                    _wait_load(b, k_win)
                    _localize(idx_v.at[b], astart + k_win * W, start, end)
                    pltpu.async_copy(rows_v.at[b], acc.at[idx_v.at[b]],
                                     scat_sem.at[b], add=True)
                    pb = (b + LA) % NB

                    @pl.when(k_win >= LA)
                    def _():
                        _wait_scat(pb)

                    @pl.when(k_win + LA < nw)
                    def _():
                        _issue_load(pb, k_win + LA)
            return carry

        lax.fori_loop(0, (nw + (NB - 1)) // NB, outer, 0)

        for b in range(NB):
            cond = jnp.zeros((), jnp.bool_)
            for dt in range(1, LA + 1):
                jt = nw - dt
                cond = jnp.logical_or(
                    cond, jnp.logical_and(jt >= 0, jt % NB == b))

            @pl.when(cond)
            def _():
                _wait_scat(b)

        rem = end - (astart + nw * W)

        @pl.when(rem > 0)
        def _remainder():
            r1 = jnp.maximum(((end + 7) // 8) * 8 - W, 0)
            r1 = pl.multiple_of(r1, 8)
            pltpu.sync_copy(idx1d_hbm.at[pl.ds(r1, W)], idx_r)
            pltpu.sync_copy(x_hbm.at[pl.ds(r1, W), :], rows_v.at[0])
            _localize(idx_r, r1, jnp.maximum(start, astart + nw * W), end)
            pltpu.sync_copy(rows_v.at[0], acc.at[idx_r], add=True)

        def _scale_out(aoff, size):
            pltpu.sync_copy(acc.at[pl.ds(arow + aoff, size), :],
                            zbuf.at[pl.ds(0, size), :])

            def scale_body(i, carry):
                zbuf[i // DC, pl.ds((i % DC) * 16, 16)] = (
                    zbuf[i // DC, pl.ds((i % DC) * 16, 16)] * SCALE)
                return carry

            lax.fori_loop(0, size * DC, scale_body, 0)
            pltpu.sync_copy(zbuf.at[pl.ds(0, size), :],
                            out_hbm.at[pl.ds(seg0 + aoff, size), :])

        for off, size in ((0, 64), (64, 64), (128, 64), (192, 64), (256, 56)):
            _scale_out(off, size)

        @pl.when(s == NS - 1)
        def _last():
            _scale_out(SEG_PER_W, SEG_LAST - SEG_PER_W)

    return k(x, idx1d, bounds)


def kernel(x, index, dim, dim_size):
    del dim, dim_size
    idx = index.astype(jnp.int32)
    wids = jnp.arange(NWORK, dtype=jnp.int32)
    targets = (wids // NS) * 5000 + (wids % NS) * SEG_PER_W
    bounds = jnp.searchsorted(idx, targets).astype(jnp.int32)
    bounds = jnp.concatenate(
        [bounds, jnp.full((48 - NWORK,), N_ROWS, dtype=jnp.int32)])
    return _segsum(x, idx, bounds)


# speedup vs baseline: 1.0974x; 1.0550x over previous
import functools

import jax
import jax.numpy as jnp
from jax import lax
from jax.experimental import pallas as pl
from jax.experimental.pallas import tpu as pltpu
from jax.experimental.pallas import tpu_sc as plsc

N_ROWS = 320000
D = 128
DC = D // 16
S = 10000
NC, NS = 2, 16
NWORK = NC * NS
SEG_PER_W = 312
SEG_LAST = 320
ACC_R = 5024
W = 64
NB = 8
LA = 5
SCALE = 1.0 / (32.0 ** 0.5)


def _segsum(x, idx1d, bounds):
    mesh = plsc.VectorSubcoreMesh(core_axis_name="c", subcore_axis_name="s")

    @functools.partial(
        pl.kernel,
        out_type=jax.ShapeDtypeStruct((S, D), jnp.float32),
        mesh=mesh,
        scratch_types=[
            pltpu.VMEM((48,), jnp.int32),
            pltpu.VMEM((NB, W), jnp.int32),
            pltpu.VMEM((W,), jnp.int32),
            pltpu.VMEM((NB, W, D), jnp.float32),
            pltpu.VMEM((64, D), jnp.float32),
            pltpu.VMEM_SHARED((ACC_R, D), jnp.float32),
            pltpu.SemaphoreType.DMA((NB,)),
            pltpu.SemaphoreType.DMA((NB,)),
        ],
    )
    def k(x_hbm, idx1d_hbm, bounds_hbm, out_hbm, bounds_v, idx_v,
          idx_r, rows_v, zbuf, acc, load_sem, scat_sem):
        c = lax.axis_index("c")
        s = lax.axis_index("s")
        wid = c * NS + s
        seg0 = c * 5000 + s * SEG_PER_W

        pltpu.sync_copy(bounds_hbm, bounds_v)
        bv = bounds_v[pl.ds(wid, 16)]
        start = bv[0]
        end = bv[1]
        astart = pl.multiple_of((start // W) * W, W)
        nw = (end - astart) // W

        zero = jnp.zeros((16,), jnp.float32)

        def zero_body(i, carry):
            zbuf[i // DC, pl.ds((i % DC) * 16, 16)] = zero
            return carry

        lax.fori_loop(0, 64 * DC, zero_body, 0)
        arow = s * SEG_PER_W
        for off, size in ((0, 64), (64, 64), (128, 64), (192, 64), (256, 56)):
            pltpu.sync_copy(zbuf.at[pl.ds(0, size), :],
                            acc.at[pl.ds(arow + off, size), :])

        @pl.when(s == NS - 1)
        def _zero_last():
            pltpu.sync_copy(zbuf.at[pl.ds(0, SEG_LAST - SEG_PER_W), :],
                            acc.at[pl.ds(arow + SEG_PER_W,
                                         SEG_LAST - SEG_PER_W), :])

        def _issue_load(b, k_win):
            r0 = astart + k_win * W
            pltpu.async_copy(idx1d_hbm.at[pl.ds(r0, W)], idx_v.at[b],
                             load_sem.at[b])
            pltpu.async_copy(x_hbm.at[pl.ds(r0, W), :], rows_v.at[b],
                             load_sem.at[b])

        def _wait_load(b, k_win):
            r0 = astart + k_win * W
            pltpu.make_async_copy(idx1d_hbm.at[pl.ds(r0, W)], idx_v.at[b],
                                  load_sem.at[b]).wait()
            pltpu.make_async_copy(x_hbm.at[pl.ds(r0, W), :], rows_v.at[b],
                                  load_sem.at[b]).wait()

        def _wait_scat(b):
            pltpu.make_async_copy(rows_v.at[b], acc.at[idx_r],
                                  scat_sem.at[b]).wait()

        def _localize(iref, w0, lo, hi):
            def group(g, carry):
                r16 = g * 16
                rgv = w0 + r16 + lax.iota(jnp.int32, 16)
                inb = jnp.logical_and(rgv >= lo, rgv < hi)
                tgtv = jnp.where(inb, iref[pl.ds(r16, 16)] - c * 5000,
                                 5000 + s)
                iref[pl.ds(r16, 16)] = tgtv
                return carry

            lax.fori_loop(0, W // 16, group, 0)

        for b in range(LA):
            @pl.when(b < nw)
            def _():
                _issue_load(b, b)

        def outer(t, carry):
            for b in range(NB):
                k_win = t * NB + b

                @pl.when(k_win < nw)
                def _():
                    _wait_load(b, k_win)
                    _localize(idx_v.at[b], astart + k_win * W, start, end)
                    pltpu.async_copy(rows_v.at[b], acc.at[idx_v.at[b]],
                                     scat_sem.at[b], add=True)
                    pb = (b + LA) % NB

                    @pl.when(k_win >= NB - LA)
                    def _():
                        _wait_scat(pb)

                    @pl.when(k_win + LA < nw)
                    def _():
                        _issue_load(pb, k_win + LA)
            return carry

        lax.fori_loop(0, (nw + (NB - 1)) // NB, outer, 0)

        for b in range(NB):
            cond = jnp.zeros((), jnp.bool_)
            for dt in range(1, (NB - LA) + 1):
                jt = nw - dt
                cond = jnp.logical_or(
                    cond, jnp.logical_and(jt >= 0, jt % NB == b))

            @pl.when(cond)
            def _():
                _wait_scat(b)

        rem = end - (astart + nw * W)

        @pl.when(rem > 0)
        def _remainder():
            r1 = jnp.maximum(((end + 7) // 8) * 8 - W, 0)
            r1 = pl.multiple_of(r1, 8)
            pltpu.sync_copy(idx1d_hbm.at[pl.ds(r1, W)], idx_r)
            pltpu.sync_copy(x_hbm.at[pl.ds(r1, W), :], rows_v.at[0])
            _localize(idx_r, r1, jnp.maximum(start, astart + nw * W), end)
            pltpu.sync_copy(rows_v.at[0], acc.at[idx_r], add=True)

        def _scale_out(aoff, size):
            pltpu.sync_copy(acc.at[pl.ds(arow + aoff, size), :],
                            zbuf.at[pl.ds(0, size), :])

            def scale_body(i, carry):
                zbuf[i // DC, pl.ds((i % DC) * 16, 16)] = (
                    zbuf[i // DC, pl.ds((i % DC) * 16, 16)] * SCALE)
                return carry

            lax.fori_loop(0, size * DC, scale_body, 0)
            pltpu.sync_copy(zbuf.at[pl.ds(0, size), :],
                            out_hbm.at[pl.ds(seg0 + aoff, size), :])

        for off, size in ((0, 64), (64, 64), (128, 64), (192, 64), (256, 56)):
            _scale_out(off, size)

        @pl.when(s == NS - 1)
        def _last():
            _scale_out(SEG_PER_W, SEG_LAST - SEG_PER_W)

    return k(x, idx1d, bounds)


def kernel(x, index, dim, dim_size):
    del dim, dim_size
    idx = index.astype(jnp.int32)
    wids = jnp.arange(NWORK, dtype=jnp.int32)
    targets = (wids // NS) * 5000 + (wids % NS) * SEG_PER_W
    bounds = jnp.searchsorted(idx, targets).astype(jnp.int32)
    bounds = jnp.concatenate(
        [bounds, jnp.full((48 - NWORK,), N_ROWS, dtype=jnp.int32)])
    return _segsum(x, idx, bounds)
